# baseline (speedup 1.0000x reference)
import functools

import jax
import jax.numpy as jnp
from jax import lax
from jax.experimental import pallas as pl
from jax.experimental.pallas import tpu as pltpu
from jax.experimental.pallas import tpu_sc as plsc

VOCAB = 1000000
HIDDEN = 64
SEQ = 200
BATCH = 4096

B = BATCH * SEQ
NC, NS, L = 2, 16, 16
NW = NC * NS
B_PER_W = B // NW
CHUNK = 512
G = CHUNK // 128
N_CHUNKS = B_PER_W // CHUNK
IDX_GROUPS = B // 128


def _positional_encoding():
    den = jnp.exp(-jnp.arange(0, HIDDEN, 2) * jnp.log(10000.0) / HIDDEN)
    pos = jnp.arange(0, SEQ)[:, jnp.newaxis]
    pe = jnp.zeros((SEQ, HIDDEN), dtype=jnp.float32)
    pe = pe.at[:, 0::2].set(jnp.sin(pos * den))
    pe = pe.at[:, 1::2].set(jnp.cos(pos * den))
    return pe


def _embed_kernel(table_hbm, idx_hbm, pe_hbm, out_hbm,
                  idx_v, rows_v, pe_v, sem, pe_sem):
    wid = lax.axis_index("s") * NC + lax.axis_index("c")

    pltpu.async_copy(pe_hbm, pe_v, pe_sem).wait()

    def chunk_body(c, carry):
        goff = wid * (B_PER_W // 128) + c * G
        row_off = wid * B_PER_W + c * CHUNK

        pltpu.sync_copy(idx_hbm.at[pl.ds(goff, G)], idx_v)

        copies = [
            pltpu.async_copy(
                table_hbm.at[idx_v.at[g]],
                rows_v.at[pl.ds(g * 128, 128)],
                sem,
            )
            for g in range(G)
        ]
        for cp in copies:
            cp.wait()

        pos0 = lax.rem(c * CHUNK, SEQ)

        def add_body(r, acc):
            p = lax.rem(pos0 + r, SEQ)
            for j in range(HIDDEN // L):
                sl = pl.ds(j * L, L)
                rows_v[r, sl] = rows_v[r, sl] + pe_v[p, sl]
            return acc

        lax.fori_loop(0, CHUNK, add_body, 0)

        pltpu.sync_copy(rows_v, out_hbm.at[pl.ds(row_off, CHUNK)])
        return carry

    lax.fori_loop(0, N_CHUNKS, chunk_body, 0)


@jax.jit
def kernel(tokens, table):
    idx = tokens.astype(jnp.int32).reshape(IDX_GROUPS, 128)
    pe = _positional_encoding()

    mesh = plsc.VectorSubcoreMesh(core_axis_name="c", subcore_axis_name="s")
    out = pl.kernel(
        _embed_kernel,
        out_type=jax.ShapeDtypeStruct((B, HIDDEN), jnp.float32),
        mesh=mesh,
        scratch_types=[
            pltpu.VMEM((G, 128), jnp.int32),
            pltpu.VMEM((CHUNK, HIDDEN), jnp.float32),
            pltpu.VMEM((SEQ, HIDDEN), jnp.float32),
            pltpu.SemaphoreType.DMA,
            pltpu.SemaphoreType.DMA,
        ],
        compiler_params=pltpu.CompilerParams(use_tc_tiling_on_sc=False),
    )(table, idx, pe)
    return out.reshape(BATCH, SEQ, HIDDEN)


# speedup vs baseline: 1.4211x; 1.4211x over previous
import jax
import jax.numpy as jnp
from jax import lax
from jax.experimental import pallas as pl
from jax.experimental.pallas import tpu as pltpu
from jax.experimental.pallas import tpu_sc as plsc

VOCAB = 1000000
HIDDEN = 64
SEQ = 200
BATCH = 4096

B = BATCH * SEQ
NC, NS, L = 2, 16, 16
NW = NC * NS
B_PER_W = B // NW
CHUNK = SEQ
GSZ = 100
G = CHUNK // GSZ
N_CHUNKS = B_PER_W // CHUNK
GRP_PER_W = B_PER_W // GSZ
NBUF = 4


def _positional_encoding():
    den = jnp.exp(-jnp.arange(0, HIDDEN, 2) * jnp.log(10000.0) / HIDDEN)
    pos = jnp.arange(0, SEQ)[:, jnp.newaxis]
    pe = jnp.zeros((SEQ, HIDDEN), dtype=jnp.float32)
    pe = pe.at[:, 0::2].set(jnp.sin(pos * den))
    pe = pe.at[:, 1::2].set(jnp.cos(pos * den))
    return pe


def _embed_kernel(table_hbm, idx_hbm, pe_hbm, out_hbm,
                  idx_v, rows0, rows1, rows2, rows3, pe_v,
                  g0, g1, g2, g3, s0, s1, s2, s3, aux_sem):
    bufs = [rows0, rows1, rows2, rows3]
    gsems = [g0, g1, g2, g3]
    ssems = [s0, s1, s2, s3]
    wid = lax.axis_index("s") * NC + lax.axis_index("c")
    row_base = wid * B_PER_W

    pltpu.async_copy(pe_hbm, pe_v, aux_sem).wait()
    pltpu.async_copy(
        idx_hbm.at[pl.ds(wid * GRP_PER_W, GRP_PER_W)], idx_v, aux_sem
    ).wait()

    def fire_gather(c, buf, sem):
        for g in range(G):
            pltpu.async_copy(
                table_hbm.at[idx_v.at[c * G + g]],
                buf.at[pl.ds(g * GSZ, GSZ)],
                sem,
            )

    def wait_gather(sem, buf):
        pltpu.make_async_copy(out_hbm.at[pl.ds(0, CHUNK)], buf, sem).wait()

    def wait_store(sem, buf):
        pltpu.make_async_copy(buf, out_hbm.at[pl.ds(0, CHUNK)], sem).wait()

    def add_pe(buf):
        @pl.loop(0, SEQ, unroll=4)
        def _(p):
            for j in range(HIDDEN // L):
                sl = pl.ds(j * L, L)
                plsc.addupdate(buf.at[p, sl], pe_v[p, sl])

    for b in range(NBUF - 1):
        fire_gather(b, bufs[b], gsems[b])

    @pl.loop(0, N_CHUNKS // NBUF)
    def _(c4):
        for b in range(NBUF):
            c = c4 * NBUF + b
            wait_gather(gsems[b], bufs[b])
            add_pe(bufs[b])
            pltpu.async_copy(
                bufs[b], out_hbm.at[pl.ds(row_base + c * CHUNK, CHUNK)],
                ssems[b],
            )
            nb = (b + NBUF - 1) % NBUF

            @pl.when(c + NBUF - 1 < N_CHUNKS)
            def _():
                @pl.when(c >= 1)
                def _():
                    wait_store(ssems[nb], bufs[nb])
                fire_gather(c + NBUF - 1, bufs[nb], gsems[nb])

    for b in range(NBUF):
        wait_store(ssems[b], bufs[b])


@jax.jit
def kernel(tokens, table):
    idx = tokens.astype(jnp.int32).reshape(B // GSZ, GSZ)
    pe = _positional_encoding()

    mesh = plsc.VectorSubcoreMesh(core_axis_name="c", subcore_axis_name="s")
    out = pl.kernel(
        _embed_kernel,
        out_type=jax.ShapeDtypeStruct((B, HIDDEN), jnp.float32),
        mesh=mesh,
        scratch_types=[
            pltpu.VMEM((GRP_PER_W, GSZ), jnp.int32),
            pltpu.VMEM((CHUNK, HIDDEN), jnp.float32),
            pltpu.VMEM((CHUNK, HIDDEN), jnp.float32),
            pltpu.VMEM((CHUNK, HIDDEN), jnp.float32),
            pltpu.VMEM((CHUNK, HIDDEN), jnp.float32),
            pltpu.VMEM((SEQ, HIDDEN), jnp.float32),
            pltpu.SemaphoreType.DMA,
            pltpu.SemaphoreType.DMA,
            pltpu.SemaphoreType.DMA,
            pltpu.SemaphoreType.DMA,
            pltpu.SemaphoreType.DMA,
            pltpu.SemaphoreType.DMA,
            pltpu.SemaphoreType.DMA,
            pltpu.SemaphoreType.DMA,
            pltpu.SemaphoreType.DMA,
        ],
        compiler_params=pltpu.CompilerParams(use_tc_tiling_on_sc=False),
    )(table, idx, pe)
    return out.reshape(BATCH, SEQ, HIDDEN)


# speedup vs baseline: 1.5210x; 1.0703x over previous
import jax
import jax.numpy as jnp
from jax import lax
from jax.experimental import pallas as pl
from jax.experimental.pallas import tpu as pltpu
from jax.experimental.pallas import tpu_sc as plsc

VOCAB = 1000000
HIDDEN = 64
SEQ = 200
BATCH = 4096

NC, NS, L = 2, 16, 16
NW = NC * NS
BB = 128
N_CHUNKS = SEQ * (BATCH // BB)
C_PER_W = N_CHUNKS // NW
SQ = SEQ // 8
JB = BATCH // BB
NBUF = 4


def _positional_encoding():
    den = jnp.exp(-jnp.arange(0, HIDDEN, 2) * jnp.log(10000.0) / HIDDEN)
    pos = jnp.arange(0, SEQ)[:, jnp.newaxis]
    pe = jnp.zeros((SEQ, HIDDEN), dtype=jnp.float32)
    pe = pe.at[:, 0::2].set(jnp.sin(pos * den))
    pe = pe.at[:, 1::2].set(jnp.cos(pos * den))
    return pe


def _embed_kernel(table_hbm, idx_hbm, pe_hbm, out_hbm,
                  idx_v, rows0, rows1, rows2, rows3, pe_v,
                  g0, g1, g2, g3, s0, s1, s2, s3, aux_sem):
    bufs = [rows0, rows1, rows2, rows3]
    gsems = [g0, g1, g2, g3]
    ssems = [s0, s1, s2, s3]
    wid = lax.axis_index("s") * NC + lax.axis_index("c")
    c_base = wid * C_PER_W

    pltpu.async_copy(pe_hbm, pe_v, aux_sem).wait()
    pltpu.async_copy(idx_hbm.at[pl.ds(c_base, C_PER_W)], idx_v, aux_sem).wait()

    def chunk_sj(k):
        cp = c_base + k
        rem = lax.rem(cp, 256)
        s = 8 * (cp // 256) + lax.rem(cp, 8)
        j = rem // 8
        return s, j

    def fire_gather(k, buf, sem):
        pltpu.async_copy(table_hbm.at[idx_v.at[k]], buf, sem)

    def wait_gather(sem, buf):
        pltpu.make_async_copy(
            out_hbm.at[0, pl.ds(0, BB), pl.ds(0, HIDDEN)], buf, sem).wait()

    def wait_store(sem, buf):
        pltpu.make_async_copy(
            buf, out_hbm.at[0, pl.ds(0, BB), pl.ds(0, HIDDEN)], sem).wait()

    def add_pe_and_store(k, buf, ssem):
        s, j = chunk_sj(k)
        pe_regs = [pe_v[s, pl.ds(u * L, L)] for u in range(HIDDEN // L)]

        @pl.loop(0, BB, unroll=8)
        def _(p):
            for u in range(HIDDEN // L):
                plsc.addupdate(buf.at[p, pl.ds(u * L, L)], pe_regs[u])

        pltpu.async_copy(
            buf, out_hbm.at[s, pl.ds(j * BB, BB), pl.ds(0, HIDDEN)], ssem)

    for b in range(NBUF - 1):
        fire_gather(b, bufs[b], gsems[b])

    @pl.loop(0, C_PER_W // NBUF)
    def _(c4):
        for b in range(NBUF):
            k = c4 * NBUF + b
            wait_gather(gsems[b], bufs[b])
            add_pe_and_store(k, bufs[b], ssems[b])
            nb = (b + NBUF - 1) % NBUF

            @pl.when(k + NBUF - 1 < C_PER_W)
            def _():
                @pl.when(k >= 1)
                def _():
                    wait_store(ssems[nb], bufs[nb])
                fire_gather(k + NBUF - 1, bufs[nb], gsems[nb])

    for b in range(NBUF):
        wait_store(ssems[b], bufs[b])


@jax.jit
def kernel(tokens, table):
    t4 = (tokens.T.astype(jnp.int32)
          .reshape(SQ, 8, JB, BB)
          .transpose(0, 2, 1, 3)
          .reshape(N_CHUNKS, BB))
    pe = _positional_encoding()

    mesh = plsc.VectorSubcoreMesh(core_axis_name="c", subcore_axis_name="s")
    out = pl.kernel(
        _embed_kernel,
        out_type=jax.ShapeDtypeStruct((SEQ, BATCH, 2 * HIDDEN), jnp.float32),
        mesh=mesh,
        scratch_types=[
            pltpu.VMEM((C_PER_W, BB), jnp.int32),
            pltpu.VMEM((BB, HIDDEN), jnp.float32),
            pltpu.VMEM((BB, HIDDEN), jnp.float32),
            pltpu.VMEM((BB, HIDDEN), jnp.float32),
            pltpu.VMEM((BB, HIDDEN), jnp.float32),
            pltpu.VMEM((SEQ, HIDDEN), jnp.float32),
            pltpu.SemaphoreType.DMA,
            pltpu.SemaphoreType.DMA,
            pltpu.SemaphoreType.DMA,
            pltpu.SemaphoreType.DMA,
            pltpu.SemaphoreType.DMA,
            pltpu.SemaphoreType.DMA,
            pltpu.SemaphoreType.DMA,
            pltpu.SemaphoreType.DMA,
            pltpu.SemaphoreType.DMA,
        ],
        compiler_params=pltpu.CompilerParams(use_tc_tiling_on_sc=False),
    )(table, t4, pe)
    return out[:, :, :HIDDEN].transpose(1, 0, 2)
